# Initial kernel scaffold; baseline (speedup 1.0000x reference)
import jax
import jax.numpy as jnp
from jax.experimental import pallas as pl


def kernel(x, batch, weight, bias, mean_scale):
    raise NotImplementedError("write your pallas kernel here")


# speedup vs baseline: 4.4909x; 4.4909x over previous
import functools

import jax
import jax.numpy as jnp
from jax import lax
from jax.experimental import pallas as pl
from jax.experimental.pallas import tpu as pltpu
from jax.experimental.pallas import tpu_sc as plsc

N = 100000
D = 128
NUM_SEG = 256
EPS = 1e-6
LANES = 16

GROUP = 128
FULL_GROUPS = N // GROUP
REM = N - FULL_GROUPS * GROUP
NW = 32
BASE_G = FULL_GROUPS // NW
EXTRA = FULL_GROUPS - BASE_G * NW
MAXG = BASE_G + 1

R = 1000
GRID = N // R


def _sc_stats(x, batch, batch_tail):
    mesh = plsc.VectorSubcoreMesh(core_axis_name="c", subcore_axis_name="s")

    @functools.partial(
        pl.kernel,
        mesh=mesh,
        out_type=[
            jax.ShapeDtypeStruct((2, NUM_SEG, D), jnp.float32),
            jax.ShapeDtypeStruct((2, NUM_SEG, D), jnp.float32),
            jax.ShapeDtypeStruct((2, NUM_SEG, D), jnp.float32),
        ],
        scratch_types=[
            pltpu.VMEM((GROUP, D), jnp.float32),
            pltpu.VMEM((MAXG, GROUP), jnp.int32),
            pltpu.VMEM((REM,), jnp.int32),
            pltpu.VMEM((GROUP, D), jnp.float32),
            pltpu.VMEM((16, D), jnp.float32),
            pltpu.VMEM_SHARED((NUM_SEG, D), jnp.float32),
            pltpu.VMEM_SHARED((NUM_SEG, D), jnp.float32),
            pltpu.VMEM_SHARED((NUM_SEG, D), jnp.float32),
        ],
    )
    def k(x_hbm, b_hbm, btail_hbm, sums_o, sqs_o, cnts_o,
          xv, idx2, idx_rem, ones, zbuf, sum_tab, sq_tab, cnt_tab):
        cid = lax.axis_index("c")
        sid = lax.axis_index("s")
        wid = cid * 16 + sid

        zero = jnp.zeros((LANES,), jnp.float32)
        one = jnp.ones((LANES,), jnp.float32)

        def zrow(r, carry):
            for j in range(D // LANES):
                zbuf[r, pl.ds(j * LANES, LANES)] = zero
            return carry

        lax.fori_loop(0, 16, zrow, 0)

        def orow(r, carry):
            for j in range(D // LANES):
                ones[r, pl.ds(j * LANES, LANES)] = one
            return carry

        lax.fori_loop(0, GROUP, orow, 0)

        pltpu.sync_copy(zbuf, sum_tab.at[pl.ds(sid * 16, 16)])
        pltpu.sync_copy(zbuf, sq_tab.at[pl.ds(sid * 16, 16)])
        pltpu.sync_copy(zbuf, cnt_tab.at[pl.ds(sid * 16, 16)])
        plsc.subcore_barrier()

        n_g = jnp.where(wid < EXTRA, MAXG, BASE_G)
        g0 = wid * BASE_G + jnp.minimum(wid, EXTRA)

        def body(t, carry):
            @pl.when(t < n_g)
            def _():
                off = (g0 + t) * GROUP
                pltpu.sync_copy(x_hbm.at[pl.ds(off, GROUP)], xv)
                pltpu.sync_copy(b_hbm.at[pl.ds(off, GROUP)], idx2.at[t])
                idxrow = idx2.at[t]
                pltpu.sync_copy(xv, sum_tab.at[idxrow], add=True)
                pltpu.sync_copy(ones, cnt_tab.at[idxrow], add=True)

                def sqrow(r, c2):
                    for j in range(D // LANES):
                        v = xv[r, pl.ds(j * LANES, LANES)]
                        xv[r, pl.ds(j * LANES, LANES)] = v * v
                    return c2

                lax.fori_loop(0, GROUP, sqrow, 0)
                pltpu.sync_copy(xv, sq_tab.at[idxrow], add=True)
            return carry

        lax.fori_loop(0, MAXG, body, 0)

        @pl.when(wid == NW - 1)
        def _():
            pltpu.sync_copy(x_hbm.at[pl.ds(FULL_GROUPS * GROUP, REM)],
                            xv.at[pl.ds(0, REM)])
            pltpu.sync_copy(btail_hbm, idx_rem)
            pltpu.sync_copy(xv.at[pl.ds(0, REM)], sum_tab.at[idx_rem], add=True)
            pltpu.sync_copy(ones.at[pl.ds(0, REM)], cnt_tab.at[idx_rem], add=True)

            def sqrow(r, c2):
                for j in range(D // LANES):
                    v = xv[r, pl.ds(j * LANES, LANES)]
                    xv[r, pl.ds(j * LANES, LANES)] = v * v
                return c2

            lax.fori_loop(0, REM, sqrow, 0)
            pltpu.sync_copy(xv.at[pl.ds(0, REM)], sq_tab.at[idx_rem], add=True)

        plsc.subcore_barrier()

        @pl.when(sid == 0)
        def _():
            pltpu.sync_copy(sum_tab, sums_o.at[cid])
            pltpu.sync_copy(sq_tab, sqs_o.at[cid])
            pltpu.sync_copy(cnt_tab, cnts_o.at[cid])

    return k(x, batch, batch_tail)


def _tc_norm(x, batch3, sums, sqs, cnts, w2, b2, ms2):
    def body(x_ref, b_ref, sums_ref, sqs_ref, cnts_ref, w_ref, bi_ref, ms_ref,
             o_ref, a_scr, bt_scr):
        i = pl.program_id(0)

        @pl.when(i == 0)
        def _():
            sums_c = sums_ref[0] + sums_ref[1]
            sqs_c = sqs_ref[0] + sqs_ref[1]
            cnt = cnts_ref[0, :, 0:1] + cnts_ref[1, :, 0:1]
            nc = jnp.maximum(cnt, 1.0)
            m = sums_c / nc
            s = ms_ref[...]
            seg_sq = sqs_c - nc * m * m * s * (2.0 - s)
            var = jnp.maximum(seg_sq, 0.0) / nc
            a = w_ref[...] * lax.rsqrt(var + EPS)
            a_scr[...] = a
            bt_scr[...] = bi_ref[...] - a * s * m

        b = b_ref[0]
        oh = (lax.broadcasted_iota(jnp.int32, (R, NUM_SEG), 1) == b)
        oh = oh.astype(jnp.float32)
        ag = jax.lax.dot(oh, a_scr[...], preferred_element_type=jnp.float32)
        bg = jax.lax.dot(oh, bt_scr[...], preferred_element_type=jnp.float32)
        o_ref[...] = x_ref[...] * ag + bg

    return pl.pallas_call(
        body,
        grid=(GRID,),
        in_specs=[
            pl.BlockSpec((R, D), lambda i: (i, 0)),
            pl.BlockSpec((1, R, 1), lambda i: (i, 0, 0)),
            pl.BlockSpec((2, NUM_SEG, D), lambda i: (0, 0, 0)),
            pl.BlockSpec((2, NUM_SEG, D), lambda i: (0, 0, 0)),
            pl.BlockSpec((2, NUM_SEG, D), lambda i: (0, 0, 0)),
            pl.BlockSpec((1, D), lambda i: (0, 0)),
            pl.BlockSpec((1, D), lambda i: (0, 0)),
            pl.BlockSpec((1, D), lambda i: (0, 0)),
        ],
        out_specs=pl.BlockSpec((R, D), lambda i: (i, 0)),
        out_shape=jax.ShapeDtypeStruct((N, D), jnp.float32),
        scratch_shapes=[
            pltpu.VMEM((NUM_SEG, D), jnp.float32),
            pltpu.VMEM((NUM_SEG, D), jnp.float32),
        ],
        compiler_params=pltpu.CompilerParams(
            dimension_semantics=("arbitrary",)),
    )(x, batch3, sums, sqs, cnts, w2, b2, ms2)


def kernel(x, batch, weight, bias, mean_scale):
    batch_tail = batch[FULL_GROUPS * GROUP:]
    sums, sqs, cnts = _sc_stats(x, batch, batch_tail)
    batch3 = batch.reshape(GRID, R, 1)
    return _tc_norm(x, batch3, sums, sqs, cnts,
                    weight.reshape(1, D), bias.reshape(1, D),
                    mean_scale.reshape(1, D))


# speedup vs baseline: 4.8078x; 1.0706x over previous
import functools

import jax
import jax.numpy as jnp
from jax import lax
from jax.experimental import pallas as pl
from jax.experimental.pallas import tpu as pltpu
from jax.experimental.pallas import tpu_sc as plsc

N = 100000
D = 128
NUM_SEG = 256
EPS = 1e-6
LANES = 16

GROUP = 128
FULL_GROUPS = N // GROUP
REM = N - FULL_GROUPS * GROUP
NW = 32
BASE_G = FULL_GROUPS // NW
EXTRA = FULL_GROUPS - BASE_G * NW
MAXG = BASE_G + 1

R = 1000
GRID = N // R


def _sc_stats(x, batch, batch_tail):
    mesh = plsc.VectorSubcoreMesh(core_axis_name="c", subcore_axis_name="s")

    @functools.partial(
        pl.kernel,
        mesh=mesh,
        out_type=[
            jax.ShapeDtypeStruct((2, NUM_SEG, D), jnp.float32),
            jax.ShapeDtypeStruct((2, NUM_SEG, D), jnp.float32),
            jax.ShapeDtypeStruct((2, NUM_SEG, D), jnp.float32),
        ],
        scratch_types=[
            pltpu.VMEM((GROUP, D), jnp.float32),
            pltpu.VMEM((GROUP, D), jnp.float32),
            pltpu.VMEM((GROUP, D), jnp.float32),
            pltpu.VMEM((GROUP, D), jnp.float32),
            pltpu.VMEM((MAXG, GROUP), jnp.int32),
            pltpu.VMEM((REM,), jnp.int32),
            pltpu.VMEM((GROUP, D), jnp.float32),
            pltpu.VMEM((16, D), jnp.float32),
            pltpu.VMEM_SHARED((NUM_SEG, D), jnp.float32),
            pltpu.VMEM_SHARED((NUM_SEG, D), jnp.float32),
            pltpu.VMEM_SHARED((NUM_SEG, D), jnp.float32),
            pltpu.SemaphoreType.DMA,
            pltpu.SemaphoreType.DMA,
            pltpu.SemaphoreType.DMA,
            pltpu.SemaphoreType.DMA,
        ],
    )
    def k(x_hbm, b_hbm, btail_hbm, sums_o, sqs_o, cnts_o,
          xv0, xv1, sqv0, sqv1, idx2, idx_rem, ones, zbuf,
          sum_tab, sq_tab, cnt_tab, lsem0, lsem1, ssem0, ssem1):
        cid = lax.axis_index("c")
        sid = lax.axis_index("s")
        wid = cid * 16 + sid

        zero = jnp.zeros((LANES,), jnp.float32)
        one = jnp.ones((LANES,), jnp.float32)

        def zrow(r, carry):
            for j in range(D // LANES):
                zbuf[r, pl.ds(j * LANES, LANES)] = zero
            return carry

        lax.fori_loop(0, 16, zrow, 0)

        def orow(r, carry):
            for j in range(D // LANES):
                ones[r, pl.ds(j * LANES, LANES)] = one
            return carry

        lax.fori_loop(0, GROUP, orow, 0)

        pltpu.sync_copy(zbuf, sum_tab.at[pl.ds(sid * 16, 16)])
        pltpu.sync_copy(zbuf, sq_tab.at[pl.ds(sid * 16, 16)])
        pltpu.sync_copy(zbuf, cnt_tab.at[pl.ds(sid * 16, 16)])
        plsc.subcore_barrier()

        n_g = jnp.where(wid < EXTRA, MAXG, BASE_G)
        g0 = wid * BASE_G + jnp.minimum(wid, EXTRA)

        def start_load(xv, lsem, t):
            off = (g0 + t) * GROUP
            pltpu.async_copy(x_hbm.at[pl.ds(off, GROUP)], xv, lsem)
            pltpu.async_copy(b_hbm.at[pl.ds(off, GROUP)], idx2.at[t], lsem)

        def wait_load(xv, lsem, t):
            pltpu.make_async_copy(x_hbm.at[pl.ds(0, GROUP)], xv, lsem).wait()
            pltpu.make_async_copy(b_hbm.at[pl.ds(0, GROUP)], idx2.at[t],
                                  lsem).wait()

        def process(xv, sqv, ssem, t):
            idxrow = idx2.at[t]
            pltpu.async_copy(xv, sum_tab.at[idxrow], ssem, add=True)
            pltpu.async_copy(ones, cnt_tab.at[idxrow], ssem, add=True)

            def sqrow(r, c2):
                for j in range(D // LANES):
                    v = xv[r, pl.ds(j * LANES, LANES)]
                    sqv[r, pl.ds(j * LANES, LANES)] = v * v
                return c2

            lax.fori_loop(0, GROUP, sqrow, 0)
            pltpu.async_copy(sqv, sq_tab.at[idxrow], ssem, add=True)

        def wait_scatters(xv, sqv, ssem):
            idxrow = idx2.at[0]
            pltpu.make_async_copy(xv, sum_tab.at[idxrow], ssem).wait()
            pltpu.make_async_copy(ones, cnt_tab.at[idxrow], ssem).wait()
            pltpu.make_async_copy(sqv, sq_tab.at[idxrow], ssem).wait()

        def step(t, cur, nxt):
            xv, sqv, lsem, ssem = cur
            xv_n, sqv_n, lsem_n, ssem_n = nxt

            @pl.when(t + 1 < n_g)
            def _():
                @pl.when(t >= 1)
                def _():
                    wait_scatters(xv_n, sqv_n, ssem_n)
                start_load(xv_n, lsem_n, t + 1)

            @pl.when(t < n_g)
            def _():
                wait_load(xv, lsem, t)
                process(xv, sqv, ssem, t)

        buf_a = (xv0, sqv0, lsem0, ssem0)
        buf_b = (xv1, sqv1, lsem1, ssem1)

        start_load(xv0, lsem0, 0)

        def pair(p, carry):
            step(2 * p, buf_a, buf_b)
            step(2 * p + 1, buf_b, buf_a)
            return carry

        lax.fori_loop(0, (MAXG + 1) // 2, pair, 0)

        wait_scatters(xv0, sqv0, ssem0)
        wait_scatters(xv1, sqv1, ssem1)

        @pl.when(wid == NW - 1)
        def _():
            pltpu.sync_copy(x_hbm.at[pl.ds(FULL_GROUPS * GROUP, REM)],
                            xv0.at[pl.ds(0, REM)])
            pltpu.sync_copy(btail_hbm, idx_rem)
            pltpu.sync_copy(xv0.at[pl.ds(0, REM)], sum_tab.at[idx_rem], add=True)
            pltpu.sync_copy(ones.at[pl.ds(0, REM)], cnt_tab.at[idx_rem], add=True)

            def sqrow(r, c2):
                for j in range(D // LANES):
                    v = xv0[r, pl.ds(j * LANES, LANES)]
                    xv0[r, pl.ds(j * LANES, LANES)] = v * v
                return c2

            lax.fori_loop(0, REM, sqrow, 0)
            pltpu.sync_copy(xv0.at[pl.ds(0, REM)], sq_tab.at[idx_rem], add=True)

        plsc.subcore_barrier()

        @pl.when(sid == 0)
        def _():
            pltpu.sync_copy(sum_tab, sums_o.at[cid])
            pltpu.sync_copy(sq_tab, sqs_o.at[cid])
            pltpu.sync_copy(cnt_tab, cnts_o.at[cid])

    return k(x, batch, batch_tail)


def _tc_norm(x, batch3, sums, sqs, cnts, w2, b2, ms2):
    def body(x_ref, b_ref, sums_ref, sqs_ref, cnts_ref, w_ref, bi_ref, ms_ref,
             o_ref, a_scr, bt_scr):
        i = pl.program_id(0)

        @pl.when(i == 0)
        def _():
            sums_c = sums_ref[0] + sums_ref[1]
            sqs_c = sqs_ref[0] + sqs_ref[1]
            cnt = cnts_ref[0, :, 0:1] + cnts_ref[1, :, 0:1]
            nc = jnp.maximum(cnt, 1.0)
            m = sums_c / nc
            s = ms_ref[...]
            seg_sq = sqs_c - nc * m * m * s * (2.0 - s)
            var = jnp.maximum(seg_sq, 0.0) / nc
            a = w_ref[...] * lax.rsqrt(var + EPS)
            a_scr[...] = a
            bt_scr[...] = bi_ref[...] - a * s * m

        b = b_ref[0]
        oh = (lax.broadcasted_iota(jnp.int32, (R, NUM_SEG), 1) == b)
        oh = oh.astype(jnp.float32)
        ag = jax.lax.dot(oh, a_scr[...], preferred_element_type=jnp.float32)
        bg = jax.lax.dot(oh, bt_scr[...], preferred_element_type=jnp.float32)
        o_ref[...] = x_ref[...] * ag + bg

    return pl.pallas_call(
        body,
        grid=(GRID,),
        in_specs=[
            pl.BlockSpec((R, D), lambda i: (i, 0)),
            pl.BlockSpec((1, R, 1), lambda i: (i, 0, 0)),
            pl.BlockSpec((2, NUM_SEG, D), lambda i: (0, 0, 0)),
            pl.BlockSpec((2, NUM_SEG, D), lambda i: (0, 0, 0)),
            pl.BlockSpec((2, NUM_SEG, D), lambda i: (0, 0, 0)),
            pl.BlockSpec((1, D), lambda i: (0, 0)),
            pl.BlockSpec((1, D), lambda i: (0, 0)),
            pl.BlockSpec((1, D), lambda i: (0, 0)),
        ],
        out_specs=pl.BlockSpec((R, D), lambda i: (i, 0)),
        out_shape=jax.ShapeDtypeStruct((N, D), jnp.float32),
        scratch_shapes=[
            pltpu.VMEM((NUM_SEG, D), jnp.float32),
            pltpu.VMEM((NUM_SEG, D), jnp.float32),
        ],
        compiler_params=pltpu.CompilerParams(
            dimension_semantics=("arbitrary",)),
    )(x, batch3, sums, sqs, cnts, w2, b2, ms2)


def kernel(x, batch, weight, bias, mean_scale):
    batch_tail = batch[FULL_GROUPS * GROUP:]
    sums, sqs, cnts = _sc_stats(x, batch, batch_tail)
    batch3 = batch.reshape(GRID, R, 1)
    return _tc_norm(x, batch3, sums, sqs, cnts,
                    weight.reshape(1, D), bias.reshape(1, D),
                    mean_scale.reshape(1, D))


# speedup vs baseline: 5.5606x; 1.1566x over previous
import functools

import jax
import jax.numpy as jnp
from jax import lax
from jax.experimental import pallas as pl
from jax.experimental.pallas import tpu as pltpu
from jax.experimental.pallas import tpu_sc as plsc

N = 100000
D = 128
NUM_SEG = 256
EPS = 1e-6
LANES = 16

GROUP = 64
FULL_GROUPS = N // GROUP
REM = N - FULL_GROUPS * GROUP
NW = 32
BASE_G = FULL_GROUPS // NW
EXTRA = FULL_GROUPS - BASE_G * NW
MAXG = BASE_G + 1

R = 1000
GRID = N // R


def _sc_stats(x, batch, batch_tail):
    mesh = plsc.VectorSubcoreMesh(core_axis_name="c", subcore_axis_name="s")

    @functools.partial(
        pl.kernel,
        mesh=mesh,
        out_type=[
            jax.ShapeDtypeStruct((NW, NUM_SEG, D), jnp.float32),
            jax.ShapeDtypeStruct((NW, NUM_SEG, D), jnp.float32),
            jax.ShapeDtypeStruct((NW, NUM_SEG, LANES), jnp.float32),
        ],
        scratch_types=[
            pltpu.VMEM((GROUP, D), jnp.float32),
            pltpu.VMEM((GROUP, D), jnp.float32),
            pltpu.VMEM((2, GROUP), jnp.int32),
            pltpu.VMEM((REM,), jnp.int32),
            pltpu.VMEM((NUM_SEG, D), jnp.float32),
            pltpu.VMEM((NUM_SEG, D), jnp.float32),
            pltpu.VMEM((NUM_SEG, LANES), jnp.float32),
            pltpu.SemaphoreType.DMA,
            pltpu.SemaphoreType.DMA,
        ],
    )
    def k(x_hbm, b_hbm, btail_hbm, sums_o, sqs_o, cnts_o,
          xv0, xv1, idx2, idx_rem, sum_v, sq_v, cnt_v, lsem0, lsem1):
        cid = lax.axis_index("c")
        sid = lax.axis_index("s")
        wid = cid * 16 + sid

        zero = jnp.zeros((LANES,), jnp.float32)

        def zrow(r, carry):
            cnt_v[r, :] = zero
            for j in range(D // LANES):
                sum_v[r, pl.ds(j * LANES, LANES)] = zero
                sq_v[r, pl.ds(j * LANES, LANES)] = zero
            return carry

        lax.fori_loop(0, NUM_SEG, zrow, 0)

        n_g = jnp.where(wid < EXTRA, MAXG, BASE_G)
        g0 = wid * BASE_G + jnp.minimum(wid, EXTRA)

        def start_load(xv, lsem, slot, t):
            off = (g0 + t) * GROUP
            pltpu.async_copy(x_hbm.at[pl.ds(off, GROUP)], xv, lsem)
            pltpu.async_copy(b_hbm.at[pl.ds(off, GROUP)], idx2.at[slot], lsem)

        def wait_load(xv, lsem, slot):
            pltpu.make_async_copy(x_hbm.at[pl.ds(0, GROUP)], xv, lsem).wait()
            pltpu.make_async_copy(b_hbm.at[pl.ds(0, GROUP)], idx2.at[slot],
                                  lsem).wait()

        def accum_row(xv, seg, r):
            plsc.addupdate(cnt_v.at[seg], jnp.ones((LANES,), jnp.float32))
            for j in range(D // LANES):
                v = xv[r, pl.ds(j * LANES, LANES)]
                plsc.addupdate(sum_v.at[seg, pl.ds(j * LANES, LANES)], v)
                plsc.addupdate(sq_v.at[seg, pl.ds(j * LANES, LANES)], v * v)

        def process(xv, slot):
            ids_head = idx2[slot, pl.ds(0, LANES)]
            ids_tail = idx2[slot, pl.ds(GROUP - LANES, LANES)]
            i0 = ids_head[0]
            i_last = ids_tail[LANES - 1]

            @pl.when(i0 == i_last)
            def _():
                def row(r, accs):
                    out = []
                    for j in range(D // LANES):
                        v = xv[r, pl.ds(j * LANES, LANES)]
                        out.append(accs[2 * j] + v)
                        out.append(accs[2 * j + 1] + v * v)
                    return tuple(out)

                init = tuple(jnp.zeros((LANES,), jnp.float32)
                             for _ in range(2 * (D // LANES)))
                accs = lax.fori_loop(0, GROUP, row, init)
                for j in range(D // LANES):
                    plsc.addupdate(sum_v.at[i0, pl.ds(j * LANES, LANES)],
                                   accs[2 * j])
                    plsc.addupdate(sq_v.at[i0, pl.ds(j * LANES, LANES)],
                                   accs[2 * j + 1])
                plsc.addupdate(cnt_v.at[i0],
                               jnp.full((LANES,), float(GROUP), jnp.float32))

            @pl.when(i0 != i_last)
            def _():
                def chunk(c, carry):
                    ids_v = idx2[slot, pl.ds(c * LANES, LANES)]
                    for l in range(LANES):
                        accum_row(xv, ids_v[l], c * LANES + l)
                    return carry

                lax.fori_loop(0, GROUP // LANES, chunk, 0)

        def step(t, cur, nxt):
            xv, lsem, slot = cur
            xv_n, lsem_n, slot_n = nxt

            @pl.when(t + 1 < n_g)
            def _():
                start_load(xv_n, lsem_n, slot_n, t + 1)

            @pl.when(t < n_g)
            def _():
                wait_load(xv, lsem, slot)
                process(xv, slot)

        start_load(xv0, lsem0, 0, 0)

        def pair(p, carry):
            step(2 * p, (xv0, lsem0, 0), (xv1, lsem1, 1))
            step(2 * p + 1, (xv1, lsem1, 1), (xv0, lsem0, 0))
            return carry

        lax.fori_loop(0, (MAXG + 1) // 2, pair, 0)

        @pl.when(wid == NW - 1)
        def _():
            pltpu.sync_copy(x_hbm.at[pl.ds(FULL_GROUPS * GROUP, REM)],
                            xv0.at[pl.ds(0, REM)])
            pltpu.sync_copy(btail_hbm, idx_rem)

            def chunk(c, carry):
                ids_v = idx_rem[pl.ds(c * LANES, LANES)]
                for l in range(LANES):
                    accum_row(xv0, ids_v[l], c * LANES + l)
                return carry

            lax.fori_loop(0, REM // LANES, chunk, 0)

        pltpu.sync_copy(sum_v, sums_o.at[wid])
        pltpu.sync_copy(sq_v, sqs_o.at[wid])
        pltpu.sync_copy(cnt_v, cnts_o.at[wid])

    return k(x, batch, batch_tail)


def _tc_norm(x, batch3, sums, sqs, cnts, w2, b2, ms2):
    def body(x_ref, b_ref, sums_ref, sqs_ref, cnts_ref, w_ref, bi_ref, ms_ref,
             o_ref, a_scr, bt_scr):
        i = pl.program_id(0)

        @pl.when(i == 0)
        def _():
            sums_c = jnp.sum(sums_ref[...], axis=0)
            sqs_c = jnp.sum(sqs_ref[...], axis=0)
            cnt = jnp.sum(cnts_ref[...], axis=0)[:, 0:1]
            nc = jnp.maximum(cnt, 1.0)
            m = sums_c / nc
            s = ms_ref[...]
            seg_sq = sqs_c - nc * m * m * s * (2.0 - s)
            var = jnp.maximum(seg_sq, 0.0) / nc
            a = w_ref[...] * lax.rsqrt(var + EPS)
            a_scr[...] = a
            bt_scr[...] = bi_ref[...] - a * s * m

        b = b_ref[0]
        oh = (lax.broadcasted_iota(jnp.int32, (R, NUM_SEG), 1) == b)
        oh = oh.astype(jnp.float32)
        ag = jax.lax.dot(oh, a_scr[...], preferred_element_type=jnp.float32)
        bg = jax.lax.dot(oh, bt_scr[...], preferred_element_type=jnp.float32)
        o_ref[...] = x_ref[...] * ag + bg

    return pl.pallas_call(
        body,
        grid=(GRID,),
        in_specs=[
            pl.BlockSpec((R, D), lambda i: (i, 0)),
            pl.BlockSpec((1, R, 1), lambda i: (i, 0, 0)),
            pl.BlockSpec((NW, NUM_SEG, D), lambda i: (0, 0, 0)),
            pl.BlockSpec((NW, NUM_SEG, D), lambda i: (0, 0, 0)),
            pl.BlockSpec((NW, NUM_SEG, LANES), lambda i: (0, 0, 0)),
            pl.BlockSpec((1, D), lambda i: (0, 0)),
            pl.BlockSpec((1, D), lambda i: (0, 0)),
            pl.BlockSpec((1, D), lambda i: (0, 0)),
        ],
        out_specs=pl.BlockSpec((R, D), lambda i: (i, 0)),
        out_shape=jax.ShapeDtypeStruct((N, D), jnp.float32),
        scratch_shapes=[
            pltpu.VMEM((NUM_SEG, D), jnp.float32),
            pltpu.VMEM((NUM_SEG, D), jnp.float32),
        ],
        compiler_params=pltpu.CompilerParams(
            dimension_semantics=("arbitrary",)),
    )(x, batch3, sums, sqs, cnts, w2, b2, ms2)


def kernel(x, batch, weight, bias, mean_scale):
    batch_tail = batch[FULL_GROUPS * GROUP:]
    sums, sqs, cnts = _sc_stats(x, batch, batch_tail)
    batch3 = batch.reshape(GRID, R, 1)
    return _tc_norm(x, batch3, sums, sqs, cnts,
                    weight.reshape(1, D), bias.reshape(1, D),
                    mean_scale.reshape(1, D))
